# Initial kernel scaffold; baseline (speedup 1.0000x reference)
import jax
import jax.numpy as jnp
from jax.experimental import pallas as pl


def kernel(cue, memory):
    raise NotImplementedError("write your pallas kernel here")


# speedup vs baseline: 7.8003x; 7.8003x over previous
import functools

import jax
import jax.numpy as jnp
from jax import lax
from jax.experimental import pallas as pl
from jax.experimental.pallas import tpu as pltpu
from jax.experimental.pallas import tpu_sc as plsc

CAP = 1_000_000
DIM = 64
NC = 2
NS = 16
NW = NC * NS
CHUNK = 512
NCHUNK = 62
GROUPS = CHUNK // 16
LAST_START = 999_424
TAIL_START = 999_936
TAIL_ROWS = CAP - TAIL_START
NEG = float("-inf")

_MESH = plsc.VectorSubcoreMesh(
    core_axis_name="c", subcore_axis_name="s", num_cores=NC, num_subcores=NS
)


@functools.partial(
    pl.kernel,
    out_type=(
        jax.ShapeDtypeStruct((NW, 16), jnp.float32),
        jax.ShapeDtypeStruct((NW, DIM), jnp.float32),
    ),
    mesh=_MESH,
    compiler_params=pltpu.CompilerParams(needs_layout_passes=False),
    scratch_types=[
        pltpu.VMEM((DIM, CHUNK), jnp.float32),
        pltpu.VMEM((DIM, CHUNK), jnp.float32),
        pltpu.VMEM((DIM, TAIL_ROWS), jnp.float32),
        pltpu.VMEM((DIM,), jnp.float32),
        pltpu.SMEM((DIM,), jnp.float32),
        pltpu.VMEM((16,), jnp.float32),
        pltpu.VMEM((DIM,), jnp.float32),
        pltpu.VMEM((CHUNK,), jnp.float32),
        pltpu.SMEM((4,), jnp.float32),
        pltpu.SMEM((2,), jnp.int32),
        pltpu.SemaphoreType.DMA,
        pltpu.SemaphoreType.DMA,
    ],
)
def _scan(cue_hbm, memt_hbm, pvals_hbm, rows_hbm,
          buf0, buf1, tbuf, cue_v, cue_s, pv_v, rowk, sims_v,
          meta_f, meta_i, sem0, sem1):
    wid = lax.axis_index("c") * NS + lax.axis_index("s")
    chunk0 = wid * NCHUNK
    iota = lax.iota(jnp.int32, 16)
    zero = jnp.zeros((16,), jnp.float32)

    pltpu.sync_copy(cue_hbm, cue_v)
    for q in range(DIM // 16):
        v = cue_v[pl.ds(q * 16, 16)]
        for i in range(16):
            cue_s[q * 16 + i] = v[i]

    def phys_start(t):
        return jnp.minimum((chunk0 + t) * CHUNK, LAST_START)

    def dma(t, buf, sem):
        return pltpu.make_async_copy(
            memt_hbm.at[:, pl.ds(phys_start(t), CHUNK)], buf, sem)

    def dot_nrm(buf, rbase):
        accs = [zero] * 4
        nrms = [zero] * 4
        for j in range(DIM):
            v = buf[j, pl.ds(rbase, 16)]
            accs[j % 4] = accs[j % 4] + v * cue_s[j]
            nrms[j % 4] = nrms[j % 4] + v * v
        acc = (accs[0] + accs[1]) + (accs[2] + accs[3])
        nrm = (nrms[0] + nrms[1]) + (nrms[2] + nrms[3])
        return acc, nrm

    def score(acc, nrm):
        return acc * jnp.abs(acc) / jnp.maximum(nrm, jnp.float32(1e-16))

    def chunk_proc(buf, t, carry):
        G1, I1, G2 = carry
        phys = phys_start(t)
        delta = (chunk0 + t) * CHUNK - phys

        def gbody(g, mx):
            lrow = g * 16 + iota
            acc, nrm = dot_nrm(buf, g * 16)
            gv = jnp.where(lrow >= delta, score(acc, nrm), NEG)
            sims_v[pl.ds(g * 16, 16)] = gv
            return jnp.maximum(mx, gv)

        max16 = lax.fori_loop(0, GROUPS, gbody, jnp.full((16,), NEG, jnp.float32))
        m_c = jnp.max(max16)
        better = m_c > G1

        @pl.when(better)
        def _slow():
            def find(g, mr):
                gv = sims_v[pl.ds(g * 16, 16)]
                return jnp.minimum(mr, jnp.where(gv == m_c, g * 16 + iota,
                                                 jnp.int32(1 << 30)))
            minrow = lax.fori_loop(0, GROUPS, find,
                                   jnp.full((16,), 1 << 30, jnp.int32))
            loc = jnp.min(minrow)

            def sec(g, s16):
                gv = sims_v[pl.ds(g * 16, 16)]
                return jnp.maximum(s16, jnp.where(g * 16 + iota == loc, NEG, gv))
            s16 = lax.fori_loop(0, GROUPS, sec, jnp.full((16,), NEG, jnp.float32))
            meta_f[0] = jnp.max(s16)
            meta_i[0] = loc
            locv = jnp.full((16,), loc, jnp.int32)
            for q in range(DIM // 16):
                rowk[pl.ds(q * 16, 16)] = plsc.load_gather(
                    buf, [q * 16 + iota, locv])

        m2 = meta_f[0]
        loc = meta_i[0]
        G2n = jnp.where(better, jnp.maximum(G1, m2), jnp.maximum(G2, m_c))
        I1n = jnp.where(better, phys + loc, I1)
        return jnp.maximum(G1, m_c), I1n, G2n

    dma(0, buf0, sem0).start()

    def pair(tt, carry):
        c0 = 2 * tt
        dma(c0 + 1, buf1, sem1).start()
        dma(c0, buf0, sem0).wait()
        carry = chunk_proc(buf0, c0, carry)
        nxt = jnp.where(c0 + 2 < NCHUNK, c0 + 2, 0)
        dma(nxt, buf0, sem0).start()
        dma(c0 + 1, buf1, sem1).wait()
        carry = chunk_proc(buf1, c0 + 1, carry)
        return carry

    G1, I1, G2 = lax.fori_loop(
        0, NCHUNK // 2, pair,
        (jnp.float32(NEG), jnp.int32(0), jnp.float32(NEG)))
    dma(0, buf0, sem0).wait()

    meta_f[1] = G1
    meta_f[2] = G2
    meta_i[1] = I1

    @pl.when(wid == NW - 1)
    def _tail():
        pltpu.make_async_copy(
            memt_hbm.at[:, pl.ds(TAIL_START, TAIL_ROWS)], tbuf, sem1).start()
        pltpu.make_async_copy(
            memt_hbm.at[:, pl.ds(TAIL_START, TAIL_ROWS)], tbuf, sem1).wait()
        g1c, i1c, g2c = meta_f[1], meta_i[1], meta_f[2]
        for g in range(TAIL_ROWS // 16):
            acc, nrm = dot_nrm(tbuf, g * 16)
            gv = score(acc, nrm)
            m = jnp.max(gv)
            lane = jnp.min(jnp.where(gv == m, iota, jnp.int32(16)))
            m2g = jnp.max(jnp.where(iota == lane, NEG, gv))
            better = m > g1c

            @pl.when(better)
            def _():
                locv = jnp.full((16,), g * 16 + lane, jnp.int32)
                for q in range(DIM // 16):
                    rowk[pl.ds(q * 16, 16)] = plsc.load_gather(
                        tbuf, [q * 16 + iota, locv])

            g2c = jnp.where(better, jnp.maximum(g1c, m2g),
                            jnp.maximum(g2c, m))
            i1c = jnp.where(better, TAIL_START + g * 16 + lane, i1c)
            g1c = jnp.maximum(g1c, m)
        meta_f[1] = g1c
        meta_f[2] = g2c
        meta_i[1] = i1c

    G1 = meta_f[1]
    G2 = meta_f[2]
    I1 = meta_i[1]

    pv = jnp.where(iota == 0, G1,
                   jnp.where(iota == 1, G2,
                             jnp.where(iota == 2, I1.astype(jnp.float32),
                                       jnp.float32(0.0))))
    pv_v[...] = pv
    pltpu.sync_copy(pv_v, pvals_hbm.at[wid])
    pltpu.sync_copy(rowk, rows_hbm.at[wid])


def kernel(cue, memory):
    pvals, rows = _scan(cue, memory.T)
    g1 = pvals[:, 0]
    g2 = pvals[:, 1]
    w = jnp.argmax(g1)
    G1 = g1[w]
    Gi = jnp.maximum(g2[w], jnp.max(g1.at[w].set(NEG)))
    norm_c = jnp.maximum(jnp.sqrt(jnp.sum(cue * cue)), 1e-8)

    def to_sim(g):
        return jnp.sign(g) * jnp.sqrt(jnp.abs(g)) / norm_c

    sim0 = to_sim(G1)
    interference = jnp.maximum(to_sim(Gi), 0.0)
    confidence = sim0 / (sim0 + interference + 1e-9)
    return rows[w], sim0, confidence
